# baseline (speedup 1.0000x reference)
import functools

import numpy as np
import jax
import jax.numpy as jnp
from jax import lax
from jax.experimental import pallas as pl
from jax.experimental.pallas import tpu as pltpu
from jax.experimental.pallas import tpu_sc as plsc


def _positional_table(T, E):
    pos = np.arange(T, dtype=np.float32)[:, None]
    i = np.arange(E, dtype=np.float32)[None, :]
    angles = pos / np.power(10000.0, 2.0 * i / E)
    table = np.array(angles, dtype=np.float32)
    table[:, 0::2] = np.sin(table[:, 0::2])
    table[:, 1::2] = np.cos(table[:, 1::2])
    return table


def kernel(x):
    B, T, E = x.shape
    R = T * E
    C = 2
    CW = C * R
    tab_rep = jnp.asarray(np.tile(_positional_table(T, E).reshape(R), C))
    xf = x.reshape(B * R)

    info = plsc.get_sparse_core_info()
    NW = info.num_cores * info.num_subcores
    rows_per_w = B // NW
    n_chunks = rows_per_w // C
    WW = rows_per_w * R

    mesh = plsc.VectorSubcoreMesh(core_axis_name="c", subcore_axis_name="s")

    @functools.partial(
        pl.kernel,
        out_type=jax.ShapeDtypeStruct((B * R,), jnp.float32),
        mesh=mesh,
        scratch_types=[
            pltpu.VMEM((CW,), jnp.float32),
            pltpu.VMEM((CW,), jnp.float32),
        ],
    )
    def sc_add(x_hbm, tab_hbm, out_hbm, buf, tabv):
        cid = lax.axis_index("c")
        sid = lax.axis_index("s")
        wid = sid * info.num_cores + cid
        base = wid * WW
        pltpu.sync_copy(tab_hbm, tabv)

        def chunk(i, carry):
            off = base + i * CW
            pltpu.sync_copy(x_hbm.at[pl.ds(off, CW)], buf)

            @plsc.parallel_loop(0, CW, step=16, unroll=8)
            def _(o):
                buf[pl.ds(o, 16)] = buf[pl.ds(o, 16)] + tabv[pl.ds(o, 16)]

            pltpu.sync_copy(buf, out_hbm.at[pl.ds(off, CW)])
            return carry

        lax.fori_loop(0, n_chunks, chunk, 0)

    out = sc_add(xf, tab_rep)
    return out.reshape(B, T, E)


# speedup vs baseline: 1.1231x; 1.1231x over previous
import functools

import numpy as np
import jax
import jax.numpy as jnp
from jax import lax
from jax.experimental import pallas as pl
from jax.experimental.pallas import tpu as pltpu
from jax.experimental.pallas import tpu_sc as plsc


def _positional_table(T, E):
    pos = np.arange(T, dtype=np.float32)[:, None]
    i = np.arange(E, dtype=np.float32)[None, :]
    angles = pos / np.power(10000.0, 2.0 * i / E)
    table = np.array(angles, dtype=np.float32)
    table[:, 0::2] = np.sin(table[:, 0::2])
    table[:, 1::2] = np.cos(table[:, 1::2])
    return table


def kernel(x):
    B, T, E = x.shape
    R = T * E
    CW = R
    tab = jnp.asarray(_positional_table(T, E).reshape(R))
    xf = x.reshape(B * R)

    info = plsc.get_sparse_core_info()
    NW = info.num_cores * info.num_subcores
    rows_per_w = B // NW
    n_chunks = rows_per_w
    WW = rows_per_w * R
    n_pairs = n_chunks // 2

    mesh = plsc.VectorSubcoreMesh(core_axis_name="c", subcore_axis_name="s")

    @functools.partial(
        pl.kernel,
        out_type=jax.ShapeDtypeStruct((B * R,), jnp.float32),
        mesh=mesh,
        scratch_types=[
            pltpu.VMEM((CW,), jnp.float32),
            pltpu.VMEM((CW,), jnp.float32),
            pltpu.VMEM((CW,), jnp.float32),
            pltpu.VMEM((CW,), jnp.float32),
            pltpu.VMEM((CW,), jnp.float32),
            pltpu.SemaphoreType.DMA,
            pltpu.SemaphoreType.DMA,
            pltpu.SemaphoreType.DMA,
            pltpu.SemaphoreType.DMA,
        ],
    )
    def sc_add(x_hbm, tab_hbm, out_hbm, in0, in1, out0, out1, tabv,
               si0, si1, so0, so1):
        cid = lax.axis_index("c")
        sid = lax.axis_index("s")
        wid = sid * info.num_cores + cid
        base = wid * WW
        ins, outs, sis, sos = (in0, in1), (out0, out1), (si0, si1), (so0, so1)
        pltpu.sync_copy(tab_hbm, tabv)

        for b in range(2):
            pltpu.async_copy(x_hbm.at[pl.ds(base + b * CW, CW)], ins[b], sis[b])

        def pair(p, carry):
            for b in range(2):
                i = 2 * p + b
                off = base + i * CW
                pltpu.make_async_copy(
                    x_hbm.at[pl.ds(off, CW)], ins[b], sis[b]).wait()

                @pl.when(p > 0)
                def _():
                    pltpu.make_async_copy(
                        outs[b], out_hbm.at[pl.ds(off - 2 * CW, CW)],
                        sos[b]).wait()

                @plsc.parallel_loop(0, CW, step=16, unroll=8)
                def _(o):
                    outs[b][pl.ds(o, 16)] = (
                        ins[b][pl.ds(o, 16)] + tabv[pl.ds(o, 16)])

                @pl.when(i + 2 < n_chunks)
                def _():
                    pltpu.async_copy(
                        x_hbm.at[pl.ds(off + 2 * CW, CW)], ins[b], sis[b])

                pltpu.async_copy(outs[b], out_hbm.at[pl.ds(off, CW)], sos[b])
            return carry

        lax.fori_loop(0, n_pairs, pair, 0)

        for b in range(2):
            off = base + (n_chunks - 2 + b) * CW
            pltpu.make_async_copy(
                outs[b], out_hbm.at[pl.ds(off, CW)], sos[b]).wait()

    out = sc_add(xf, tab)
    return out.reshape(B, T, E)


# speedup vs baseline: 2.3107x; 2.0574x over previous
import functools

import numpy as np
import jax
import jax.numpy as jnp
from jax import lax
from jax.experimental import pallas as pl
from jax.experimental.pallas import tpu as pltpu
from jax.experimental.pallas import tpu_sc as plsc


def _positional_table(T, E):
    pos = np.arange(T, dtype=np.float32)[:, None]
    i = np.arange(E, dtype=np.float32)[None, :]
    angles = pos / np.power(10000.0, 2.0 * i / E)
    table = np.array(angles, dtype=np.float32)
    table[:, 0::2] = np.sin(table[:, 0::2])
    table[:, 1::2] = np.cos(table[:, 1::2])
    return table


def kernel(x):
    B, T, E = x.shape
    R = T * E
    CW = R
    tab = jnp.asarray(_positional_table(T, E).reshape(R))
    xf = x.reshape(B, R)

    info = plsc.get_sparse_core_info()
    NW = info.num_cores * info.num_subcores
    rows_per_w = B // NW
    n_chunks = rows_per_w
    WW = rows_per_w * R
    n_pairs = n_chunks // 2

    mesh = plsc.VectorSubcoreMesh(core_axis_name="c", subcore_axis_name="s")

    @functools.partial(
        pl.kernel,
        out_type=jax.ShapeDtypeStruct((B, R), jnp.float32),
        mesh=mesh,
        scratch_types=[
            pltpu.VMEM((CW,), jnp.float32),
            pltpu.VMEM((CW,), jnp.float32),
            pltpu.VMEM((CW,), jnp.float32),
            pltpu.VMEM((CW,), jnp.float32),
            pltpu.VMEM((CW,), jnp.float32),
            pltpu.SemaphoreType.DMA,
            pltpu.SemaphoreType.DMA,
            pltpu.SemaphoreType.DMA,
            pltpu.SemaphoreType.DMA,
        ],
    )
    def sc_add(x_hbm, tab_hbm, out_hbm, in0, in1, out0, out1, tabv,
               si0, si1, so0, so1):
        cid = lax.axis_index("c")
        sid = lax.axis_index("s")
        wid = sid * info.num_cores + cid
        base = wid * rows_per_w
        ins, outs, sis, sos = (in0, in1), (out0, out1), (si0, si1), (so0, so1)
        pltpu.sync_copy(tab_hbm, tabv)

        for b in range(2):
            pltpu.async_copy(x_hbm.at[base + b], ins[b], sis[b])

        def pair(p, carry):
            for b in range(2):
                i = 2 * p + b
                row = base + i
                pltpu.make_async_copy(x_hbm.at[row], ins[b], sis[b]).wait()

                @pl.when(p > 0)
                def _():
                    pltpu.make_async_copy(
                        outs[b], out_hbm.at[row - 2], sos[b]).wait()

                @plsc.parallel_loop(0, CW, step=16, unroll=8)
                def _(o):
                    outs[b][pl.ds(o, 16)] = (
                        ins[b][pl.ds(o, 16)] + tabv[pl.ds(o, 16)])

                @pl.when(i + 2 < n_chunks)
                def _():
                    pltpu.async_copy(x_hbm.at[row + 2], ins[b], sis[b])

                pltpu.async_copy(outs[b], out_hbm.at[row], sos[b])
            return carry

        lax.fori_loop(0, n_pairs, pair, 0)

        for b in range(2):
            row = base + n_chunks - 2 + b
            pltpu.make_async_copy(outs[b], out_hbm.at[row], sos[b]).wait()

    out = sc_add(xf, tab)
    return out.reshape(B, T, E)


# speedup vs baseline: 9.4660x; 4.0966x over previous
import numpy as np
import jax
import jax.numpy as jnp
from jax.experimental import pallas as pl
from jax.experimental.pallas import tpu as pltpu


def _positional_table(T, E):
    pos = np.arange(T, dtype=np.float32)[:, None]
    i = np.arange(E, dtype=np.float32)[None, :]
    angles = pos / np.power(10000.0, 2.0 * i / E)
    table = np.array(angles, dtype=np.float32)
    table[:, 0::2] = np.sin(table[:, 0::2])
    table[:, 1::2] = np.cos(table[:, 1::2])
    return table


def _add_kernel(x_ref, t_ref, o_ref):
    o_ref[...] = x_ref[...] + t_ref[...]


def kernel(x):
    B, T, E = x.shape
    TE = T * E
    tab_col = jnp.asarray(_positional_table(T, E).reshape(TE, 1))
    xt = x.reshape(B, TE).T
    BS = 128
    out = pl.pallas_call(
        _add_kernel,
        grid=(B // BS,),
        in_specs=[
            pl.BlockSpec((TE, BS), lambda i: (0, i)),
            pl.BlockSpec((TE, 1), lambda i: (0, 0)),
        ],
        out_specs=pl.BlockSpec((TE, BS), lambda i: (0, i)),
        out_shape=jax.ShapeDtypeStruct((TE, B), x.dtype),
        compiler_params=pltpu.CompilerParams(
            dimension_semantics=("arbitrary",),
        ),
    )(xt, tab_col)
    return out.T.reshape(B, T, E)
